# Initial kernel scaffold; baseline (speedup 1.0000x reference)
import jax
import jax.numpy as jnp
from jax.experimental import pallas as pl


def kernel(x, edge_index, att, Wcz, bcz, Wcr, bcr, Wch, bch, Wlz, blz, Wlr, blr, Wlh, blh, W1, b1, W2, b2):
    raise NotImplementedError("write your pallas kernel here")


# speedup vs baseline: 62.5486x; 62.5486x over previous
import jax
import jax.numpy as jnp
from jax import lax
from jax.experimental import pallas as pl
from jax.experimental.pallas import tpu as pltpu
from jax.experimental.pallas import tpu_sc as plsc

N = 10000
E = 160000
F_IN = 32
D = 256
PERIODS = 8
FP = F_IN * PERIODS

NC = 2
NS = 16
NW = NC * NS
NPAD = 10240
NROW128 = NPAD // 128
EB = 128
NB_W = 40
EPW = NB_W * EB
EPAD = NW * EPW
TRASH = N + 100
RPT = NPAD // NS
BLK = 256
GRID = NPAD // BLK


def _zero2d(ref, nrows):
    def body(r, c):
        for j in range(8):
            ref[r, pl.ds(j * 16, 16)] = jnp.zeros((16,), jnp.float32)
        return c
    lax.fori_loop(0, nrows, body, 0)


def _deg_body(dst_hbm, deg_out, hist, dstbuf, zrows, rowidx, acc):
    cid = lax.axis_index("c")
    sid = lax.axis_index("s")
    w = cid * NS + sid
    _zero2d(hist, NROW128)
    pltpu.sync_copy(dst_hbm.at[pl.ds(w * NB_W, NB_W)], dstbuf)

    ones = jnp.ones((16,), jnp.float32)

    def hb(j, c):
        r = j >> 3
        k = j & 7
        idx = dstbuf[r, pl.ds(k * 16, 16)]
        plsc.addupdate_scatter(hist, [idx >> 7, idx & 127], ones)
        return c
    lax.fori_loop(0, NB_W * 8, hb, 0)

    _zero2d(zrows, 5)
    pltpu.sync_copy(zrows, acc.at[pl.ds(sid * 5, 5)])
    plsc.subcore_barrier()

    def ri(i, c):
        rowidx[pl.ds(i * 16, 16)] = lax.iota(jnp.int32, 16) + i * 16
        return c
    lax.fori_loop(0, NROW128 // 16, ri, 0)
    pltpu.sync_copy(hist, acc.at[rowidx], add=True)
    plsc.subcore_barrier()

    @pl.when(sid == 0)
    def _():
        pltpu.sync_copy(acc, hist)
        pltpu.sync_copy(hist, deg_out.at[cid])


def _agg_body(src_hbm, dst_hbm, xs0, xs1, y0_out, y1_out,
              srcbuf, dstbuf, rows, zbuf, sem, acc):
    cid = lax.axis_index("c")
    sid = lax.axis_index("s")
    w = cid * NS + sid
    pltpu.sync_copy(src_hbm.at[pl.ds(w * NB_W, NB_W)], srcbuf)
    pltpu.sync_copy(dst_hbm.at[pl.ds(w * NB_W, NB_W)], dstbuf)
    _zero2d(zbuf, 128)
    base = sid * RPT

    for chunk in range(2):
        xs = (xs0, xs1)[chunk]
        yout = (y0_out, y1_out)[chunk]
        for j in range(RPT // 128):
            pltpu.sync_copy(zbuf, acc.at[pl.ds(base + j * 128, 128)])
        plsc.subcore_barrier()

        def bb(b, c):
            pltpu.async_copy(xs.at[srcbuf.at[b]], rows, sem).wait()
            pltpu.sync_copy(rows, acc.at[dstbuf.at[b]], add=True)
            return c
        lax.fori_loop(0, NB_W, bb, 0)
        plsc.subcore_barrier()

        for j in range(RPT // 128):
            pltpu.sync_copy(acc.at[pl.ds(base + j * 128, 128)], rows)
            pltpu.sync_copy(rows, yout.at[cid, pl.ds(base + j * 128, 128)])
        plsc.subcore_barrier()


def _prep_body(d0, d1, x_ref, xs0, xs1, dinv_ref):
    deg = d0[...] + d1[...] + 1.0
    dinv = lax.rsqrt(deg)
    xs = x_ref[...] * dinv
    xs0[...] = xs[:, :128]
    xs1[...] = xs[:, 128:]
    dinv_ref[...] = dinv


def _dense_body(y0, y1, xs0, xs1, dinv, wbd, bias, attr, w1, w2, b1, b2, out_ref):
    i = pl.program_id(0)
    yl = y0[0] + y0[1] + xs0[...]
    yr = y1[0] + y1[1] + xs1[...]
    y = jnp.concatenate([yl, yr], axis=1) * dinv[...]
    g = jnp.dot(y, wbd[...], preferred_element_type=jnp.float32) + bias[...]
    z = jax.nn.sigmoid(g[:, :PERIODS * D])
    t = jnp.tanh(g[:, PERIODS * D:])
    p = jax.nn.softmax(attr[...], axis=1)
    h = jnp.zeros((BLK, D), jnp.float32)
    for tt in range(PERIODS):
        pt = p[:, tt:tt + 1]
        h = h + pt * (1.0 - z[:, tt * D:(tt + 1) * D]) * t[:, tt * D:(tt + 1) * D]
    v = jnp.dot(jnp.maximum(h, 0.0), w1[...], preferred_element_type=jnp.float32)
    part = jnp.sum((v + b1[...]) * w2[...])

    @pl.when(i == 0)
    def _():
        out_ref[...] = b2[...]
    out_ref[...] += part


_sc_mesh = dict(core_axis_name="c", subcore_axis_name="s",
                num_cores=NC, num_subcores=NS)
_sc_params = pltpu.CompilerParams(needs_layout_passes=False)


def _deg_call(dstp):
    return pl.kernel(
        _deg_body,
        out_type=jax.ShapeDtypeStruct((NC, NROW128, 128), jnp.float32),
        mesh=plsc.VectorSubcoreMesh(**_sc_mesh),
        compiler_params=_sc_params,
        scratch_types=[
            pltpu.VMEM((NROW128, 128), jnp.float32),
            pltpu.VMEM((NB_W, EB), jnp.int32),
            pltpu.VMEM((5, 128), jnp.float32),
            pltpu.VMEM((NROW128,), jnp.int32),
            pltpu.VMEM_SHARED((NROW128, 128), jnp.float32),
        ],
    )(dstp)


def _agg_call(srcp, dstp, xs0, xs1):
    return pl.kernel(
        _agg_body,
        out_type=(
            jax.ShapeDtypeStruct((NC, NPAD, 128), jnp.float32),
            jax.ShapeDtypeStruct((NC, NPAD, 128), jnp.float32),
        ),
        mesh=plsc.VectorSubcoreMesh(**_sc_mesh),
        compiler_params=_sc_params,
        scratch_types=[
            pltpu.VMEM((NB_W, EB), jnp.int32),
            pltpu.VMEM((NB_W, EB), jnp.int32),
            pltpu.VMEM((EB, 128), jnp.float32),
            pltpu.VMEM((128, 128), jnp.float32),
            pltpu.SemaphoreType.DMA,
            pltpu.VMEM_SHARED((NPAD, 128), jnp.float32),
        ],
    )(srcp, dstp, xs0, xs1)


def kernel(x, edge_index, att, Wcz, bcz, Wcr, bcr, Wch, bch,
           Wlz, blz, Wlr, blr, Wlh, blh, W1, b1, W2, b2):
    f32 = jnp.float32
    X = x.reshape(N, FP)
    Xp = jnp.pad(X, ((0, NPAD - N), (0, 0)))
    pad_e = EPAD - E
    srcp = jnp.concatenate(
        [edge_index[0], jnp.zeros((pad_e,), jnp.int32)]).reshape(NW * NB_W, EB)
    dstp = jnp.concatenate(
        [edge_index[1], jnp.full((pad_e,), TRASH, jnp.int32)]).reshape(NW * NB_W, EB)

    Wz_eff = Wcz @ Wlz[:D]
    Wh_eff = Wch @ Wlh[:D]
    eye = jnp.eye(PERIODS, dtype=f32)
    WzBD = jnp.einsum("fd,pt->fptd", Wz_eff, eye).reshape(FP, PERIODS * D)
    WhBD = jnp.einsum("fd,pt->fptd", Wh_eff, eye).reshape(FP, PERIODS * D)
    WBD = jnp.concatenate([WzBD, WhBD], axis=1)
    bz = bcz @ Wlz[:D] + blz
    bh = bch @ Wlh[:D] + blh
    bias = jnp.concatenate([jnp.tile(bz, PERIODS),
                            jnp.tile(bh, PERIODS)]).reshape(1, 2 * PERIODS * D)
    W2p = jnp.pad(W2, ((0, NPAD - N), (0, 0)))
    attr = att.reshape(1, PERIODS)
    b1r = b1.reshape(1, 1)
    b2r = b2.reshape(1, 1)

    degp = _deg_call(dstp)
    d0 = degp[0].reshape(NPAD, 1)
    d1 = degp[1].reshape(NPAD, 1)

    xs0, xs1, dinv = pl.pallas_call(
        _prep_body,
        grid=(GRID,),
        in_specs=[
            pl.BlockSpec((BLK, 1), lambda i: (i, 0)),
            pl.BlockSpec((BLK, 1), lambda i: (i, 0)),
            pl.BlockSpec((BLK, FP), lambda i: (i, 0)),
        ],
        out_specs=[
            pl.BlockSpec((BLK, 128), lambda i: (i, 0)),
            pl.BlockSpec((BLK, 128), lambda i: (i, 0)),
            pl.BlockSpec((BLK, 1), lambda i: (i, 0)),
        ],
        out_shape=[
            jax.ShapeDtypeStruct((NPAD, 128), f32),
            jax.ShapeDtypeStruct((NPAD, 128), f32),
            jax.ShapeDtypeStruct((NPAD, 1), f32),
        ],
    )(d0, d1, Xp)

    y0, y1 = _agg_call(srcp, dstp, xs0, xs1)

    out = pl.pallas_call(
        _dense_body,
        grid=(GRID,),
        in_specs=[
            pl.BlockSpec((NC, BLK, 128), lambda i: (0, i, 0)),
            pl.BlockSpec((NC, BLK, 128), lambda i: (0, i, 0)),
            pl.BlockSpec((BLK, 128), lambda i: (i, 0)),
            pl.BlockSpec((BLK, 128), lambda i: (i, 0)),
            pl.BlockSpec((BLK, 1), lambda i: (i, 0)),
            pl.BlockSpec((FP, 4 * PERIODS * D // 2), lambda i: (0, 0)),
            pl.BlockSpec((1, 2 * PERIODS * D), lambda i: (0, 0)),
            pl.BlockSpec((1, PERIODS), lambda i: (0, 0)),
            pl.BlockSpec((D, 1), lambda i: (0, 0)),
            pl.BlockSpec((BLK, 1), lambda i: (i, 0)),
            pl.BlockSpec((1, 1), lambda i: (0, 0)),
            pl.BlockSpec((1, 1), lambda i: (0, 0)),
        ],
        out_specs=pl.BlockSpec((1, 1), lambda i: (0, 0)),
        out_shape=jax.ShapeDtypeStruct((1, 1), f32),
    )(y0, y1, xs0, xs1, dinv, WBD, bias, attr, W1, W2p, b1r, b2r)

    return out.reshape(-1)


# speedup vs baseline: 62.8402x; 1.0047x over previous
import jax
import jax.numpy as jnp
from jax import lax
from jax.experimental import pallas as pl
from jax.experimental.pallas import tpu as pltpu
from jax.experimental.pallas import tpu_sc as plsc

N = 10000
E = 160000
F_IN = 32
D = 256
PERIODS = 8
FP = F_IN * PERIODS

NC = 2
NS = 16
NW = NC * NS
NPAD = 10240
NROW128 = NPAD // 128
EB = 128
NB_W = 40
EPW = NB_W * EB
EPAD = NW * EPW
TRASH = N + 100
RPT = NPAD // NS
BLK = 256
GRID = NPAD // BLK


def _zero2d(ref, nrows):
    def body(r, c):
        for j in range(8):
            ref[r, pl.ds(j * 16, 16)] = jnp.zeros((16,), jnp.float32)
        return c
    lax.fori_loop(0, nrows, body, 0)


def _deg_body(dst_hbm, deg_out, hist, dstbuf, zrows, rowidx, acc):
    cid = lax.axis_index("c")
    sid = lax.axis_index("s")
    w = cid * NS + sid
    _zero2d(hist, NROW128)
    pltpu.sync_copy(dst_hbm.at[pl.ds(w * NB_W, NB_W)], dstbuf)

    ones = jnp.ones((16,), jnp.float32)

    def hb(j, c):
        r = j >> 3
        k = j & 7
        idx = dstbuf[r, pl.ds(k * 16, 16)]
        plsc.addupdate_scatter(hist, [idx >> 7, idx & 127], ones)
        return c
    lax.fori_loop(0, NB_W * 8, hb, 0)

    _zero2d(zrows, 5)
    pltpu.sync_copy(zrows, acc.at[pl.ds(sid * 5, 5)])
    plsc.subcore_barrier()

    def ri(i, c):
        rowidx[pl.ds(i * 16, 16)] = lax.iota(jnp.int32, 16) + i * 16
        return c
    lax.fori_loop(0, NROW128 // 16, ri, 0)
    pltpu.sync_copy(hist, acc.at[rowidx], add=True)
    plsc.subcore_barrier()

    @pl.when(sid == 0)
    def _():
        pltpu.sync_copy(acc, hist)
        pltpu.sync_copy(hist, deg_out.at[cid])


def _agg_body(src_hbm, dst_hbm, xs0, xs1, y0_out, y1_out,
              srcbuf, dstbuf, rows, zbuf, sem, acc):
    cid = lax.axis_index("c")
    sid = lax.axis_index("s")
    w = cid * NS + sid
    pltpu.sync_copy(src_hbm.at[pl.ds(w * NB_W, NB_W)], srcbuf)
    pltpu.sync_copy(dst_hbm.at[pl.ds(w * NB_W, NB_W)], dstbuf)
    _zero2d(zbuf, 128)
    base = sid * RPT

    for chunk in range(2):
        xs = (xs0, xs1)[chunk]
        yout = (y0_out, y1_out)[chunk]
        for j in range(RPT // 128):
            pltpu.sync_copy(zbuf, acc.at[pl.ds(base + j * 128, 128)])
        plsc.subcore_barrier()

        def bb(b, c):
            pltpu.async_copy(xs.at[srcbuf.at[b]], rows, sem).wait()
            pltpu.sync_copy(rows, acc.at[dstbuf.at[b]], add=True)
            return c
        lax.fori_loop(0, NB_W, bb, 0)
        plsc.subcore_barrier()

        for j in range(RPT // 128):
            pltpu.sync_copy(acc.at[pl.ds(base + j * 128, 128)], rows)
            pltpu.sync_copy(rows, yout.at[cid, pl.ds(base + j * 128, 128)])
        plsc.subcore_barrier()


def _prep_body(d0, d1, x_ref, xs0, xs1, dinv_ref):
    deg = d0[...] + d1[...] + 1.0
    dinv = lax.rsqrt(deg)
    xs = x_ref[...] * dinv
    xs0[...] = xs[:, :128]
    xs1[...] = xs[:, 128:]
    dinv_ref[...] = dinv


def _dense_body(y0, y1, xs0, xs1, dinv, wbd, bias, attr, w1, w2, b1, b2, out_ref):
    i = pl.program_id(0)
    yl = y0[0] + y0[1] + xs0[...]
    yr = y1[0] + y1[1] + xs1[...]
    y = jnp.concatenate([yl, yr], axis=1) * dinv[...]
    g = jnp.dot(y, wbd[...], preferred_element_type=jnp.float32) + bias[...]
    z = jax.nn.sigmoid(g[:, :PERIODS * D])
    t = jnp.tanh(g[:, PERIODS * D:])
    p = jax.nn.softmax(attr[...], axis=1)
    h = jnp.zeros((BLK, D), jnp.float32)
    for tt in range(PERIODS):
        pt = p[:, tt:tt + 1]
        h = h + pt * (1.0 - z[:, tt * D:(tt + 1) * D]) * t[:, tt * D:(tt + 1) * D]
    v = jnp.dot(jnp.maximum(h, 0.0), w1[...], preferred_element_type=jnp.float32)
    part = jnp.sum((v + b1[...]) * w2[...])

    @pl.when(i == 0)
    def _():
        out_ref[...] = b2[...]
    out_ref[...] += part


_sc_mesh = dict(core_axis_name="c", subcore_axis_name="s",
                num_cores=NC, num_subcores=NS)
_sc_params = pltpu.CompilerParams(needs_layout_passes=False)


def _deg_call(dstp):
    return pl.kernel(
        _deg_body,
        out_type=jax.ShapeDtypeStruct((NC, NROW128, 128), jnp.float32),
        mesh=plsc.VectorSubcoreMesh(**_sc_mesh),
        compiler_params=_sc_params,
        scratch_types=[
            pltpu.VMEM((NROW128, 128), jnp.float32),
            pltpu.VMEM((NB_W, EB), jnp.int32),
            pltpu.VMEM((5, 128), jnp.float32),
            pltpu.VMEM((NROW128,), jnp.int32),
            pltpu.VMEM_SHARED((NROW128, 128), jnp.float32),
        ],
    )(dstp)


def _agg_call(srcp, dstp, xs0, xs1):
    return pl.kernel(
        _agg_body,
        out_type=(
            jax.ShapeDtypeStruct((NC, NPAD, 128), jnp.float32),
            jax.ShapeDtypeStruct((NC, NPAD, 128), jnp.float32),
        ),
        mesh=plsc.VectorSubcoreMesh(**_sc_mesh),
        compiler_params=_sc_params,
        scratch_types=[
            pltpu.VMEM((NB_W, EB), jnp.int32),
            pltpu.VMEM((NB_W, EB), jnp.int32),
            pltpu.VMEM((EB, 128), jnp.float32),
            pltpu.VMEM((128, 128), jnp.float32),
            pltpu.SemaphoreType.DMA,
            pltpu.VMEM_SHARED((NPAD, 128), jnp.float32),
        ],
    )(srcp, dstp, xs0, xs1)


def kernel(x, edge_index, att, Wcz, bcz, Wcr, bcr, Wch, bch,
           Wlz, blz, Wlr, blr, Wlh, blh, W1, b1, W2, b2):
    f32 = jnp.float32
    X = x.reshape(N, FP)
    Xp = jnp.pad(X, ((0, NPAD - N), (0, 0)))
    pad_e = EPAD - E
    srcp = jnp.concatenate(
        [edge_index[0], jnp.zeros((pad_e,), jnp.int32)]).reshape(NW * NB_W, EB)
    trash = N + (jnp.arange(pad_e, dtype=jnp.int32) % (NPAD - N))
    dstp = jnp.concatenate([edge_index[1], trash]).reshape(NW * NB_W, EB)

    Wz_eff = Wcz @ Wlz[:D]
    Wh_eff = Wch @ Wlh[:D]
    eye = jnp.eye(PERIODS, dtype=f32)
    WzBD = jnp.einsum("fd,pt->fptd", Wz_eff, eye).reshape(FP, PERIODS * D)
    WhBD = jnp.einsum("fd,pt->fptd", Wh_eff, eye).reshape(FP, PERIODS * D)
    WBD = jnp.concatenate([WzBD, WhBD], axis=1)
    bz = bcz @ Wlz[:D] + blz
    bh = bch @ Wlh[:D] + blh
    bias = jnp.concatenate([jnp.tile(bz, PERIODS),
                            jnp.tile(bh, PERIODS)]).reshape(1, 2 * PERIODS * D)
    W2p = jnp.pad(W2, ((0, NPAD - N), (0, 0)))
    attr = att.reshape(1, PERIODS)
    b1r = b1.reshape(1, 1)
    b2r = b2.reshape(1, 1)

    degp = _deg_call(dstp)
    d0 = degp[0].reshape(NPAD, 1)
    d1 = degp[1].reshape(NPAD, 1)

    xs0, xs1, dinv = pl.pallas_call(
        _prep_body,
        grid=(GRID,),
        in_specs=[
            pl.BlockSpec((BLK, 1), lambda i: (i, 0)),
            pl.BlockSpec((BLK, 1), lambda i: (i, 0)),
            pl.BlockSpec((BLK, FP), lambda i: (i, 0)),
        ],
        out_specs=[
            pl.BlockSpec((BLK, 128), lambda i: (i, 0)),
            pl.BlockSpec((BLK, 128), lambda i: (i, 0)),
            pl.BlockSpec((BLK, 1), lambda i: (i, 0)),
        ],
        out_shape=[
            jax.ShapeDtypeStruct((NPAD, 128), f32),
            jax.ShapeDtypeStruct((NPAD, 128), f32),
            jax.ShapeDtypeStruct((NPAD, 1), f32),
        ],
    )(d0, d1, Xp)

    y0, y1 = _agg_call(srcp, dstp, xs0, xs1)

    out = pl.pallas_call(
        _dense_body,
        grid=(GRID,),
        in_specs=[
            pl.BlockSpec((NC, BLK, 128), lambda i: (0, i, 0)),
            pl.BlockSpec((NC, BLK, 128), lambda i: (0, i, 0)),
            pl.BlockSpec((BLK, 128), lambda i: (i, 0)),
            pl.BlockSpec((BLK, 128), lambda i: (i, 0)),
            pl.BlockSpec((BLK, 1), lambda i: (i, 0)),
            pl.BlockSpec((FP, 4 * PERIODS * D // 2), lambda i: (0, 0)),
            pl.BlockSpec((1, 2 * PERIODS * D), lambda i: (0, 0)),
            pl.BlockSpec((1, PERIODS), lambda i: (0, 0)),
            pl.BlockSpec((D, 1), lambda i: (0, 0)),
            pl.BlockSpec((BLK, 1), lambda i: (i, 0)),
            pl.BlockSpec((1, 1), lambda i: (0, 0)),
            pl.BlockSpec((1, 1), lambda i: (0, 0)),
        ],
        out_specs=pl.BlockSpec((1, 1), lambda i: (0, 0)),
        out_shape=jax.ShapeDtypeStruct((1, 1), f32),
    )(y0, y1, xs0, xs1, dinv, WBD, bias, attr, W1, W2p, b1r, b2r)

    return out.reshape(-1)


# speedup vs baseline: 67.3383x; 1.0716x over previous
import jax
import jax.numpy as jnp
from jax import lax
from jax.experimental import pallas as pl
from jax.experimental.pallas import tpu as pltpu
from jax.experimental.pallas import tpu_sc as plsc

N = 10000
E = 160000
F_IN = 32
D = 256
PERIODS = 8
FP = F_IN * PERIODS

NC = 2
NS = 16
NW = NC * NS
NPAD = 10240
NROW128 = NPAD // 128
EB = 128
NB_W = 40
EPW = NB_W * EB
EPAD = NW * EPW
TRASH = N + 100
RPT = NPAD // NS
BLK = 256
GRID = NPAD // BLK


def _zero2d(ref, nrows):
    def body(r, c):
        for j in range(8):
            ref[r, pl.ds(j * 16, 16)] = jnp.zeros((16,), jnp.float32)
        return c
    lax.fori_loop(0, nrows, body, 0)


def _deg_body(dst_hbm, deg_out, hist, dstbuf, zrows, rowidx, acc):
    cid = lax.axis_index("c")
    sid = lax.axis_index("s")
    w = cid * NS + sid
    _zero2d(hist, NROW128)
    pltpu.sync_copy(dst_hbm.at[pl.ds(w * NB_W, NB_W)], dstbuf)

    ones = jnp.ones((16,), jnp.float32)

    def hb(j, c):
        r = j >> 3
        k = j & 7
        idx = dstbuf[r, pl.ds(k * 16, 16)]
        plsc.addupdate_scatter(hist, [idx >> 7, idx & 127], ones)
        return c
    lax.fori_loop(0, NB_W * 8, hb, 0)

    _zero2d(zrows, 5)
    pltpu.sync_copy(zrows, acc.at[pl.ds(sid * 5, 5)])
    plsc.subcore_barrier()

    def ri(i, c):
        rowidx[pl.ds(i * 16, 16)] = lax.iota(jnp.int32, 16) + i * 16
        return c
    lax.fori_loop(0, NROW128 // 16, ri, 0)
    pltpu.sync_copy(hist, acc.at[rowidx], add=True)
    plsc.subcore_barrier()

    @pl.when(sid == 0)
    def _():
        pltpu.sync_copy(acc, hist)
        pltpu.sync_copy(hist, deg_out.at[cid])


NBUF = 2


def _agg_body(src_hbm, dst_hbm, xs0, xs1, y0_out, y1_out,
              srcbuf, dstbuf, r0, r1, trashidx,
              g0, g1, s0, s1, acc):
    rows = (r0, r1)
    gsem = (g0, g1)
    ssem = (s0, s1)
    cid = lax.axis_index("c")
    sid = lax.axis_index("s")
    w = cid * NS + sid
    pltpu.sync_copy(src_hbm.at[pl.ds(w * NB_W, NB_W)], srcbuf)
    pltpu.sync_copy(dst_hbm.at[pl.ds(w * NB_W, NB_W)], dstbuf)

    def ti(i, c):
        trashidx[0, pl.ds(i * 16, 16)] = lax.iota(jnp.int32, 16) + (N + i * 16)
        return c
    lax.fori_loop(0, 8, ti, 0)
    base = sid * RPT

    for chunk in range(2):
        xs = (xs0, xs1)[chunk]
        yout = (y0_out, y1_out)[chunk]
        _zero2d(r0, 128)
        for j in range(RPT // 128):
            pltpu.sync_copy(r0, acc.at[pl.ds(base + j * 128, 128)])
        plsc.subcore_barrier()

        for j in range(NBUF):
            pltpu.async_copy(rows[j], acc.at[trashidx.at[0]], ssem[j], add=True)

        def quad(q, c):
            b = q * NBUF
            cps = []
            for j in range(NBUF):
                pltpu.make_async_copy(xs.at[pl.ds(0, EB)], rows[j], ssem[j]).wait()
                cps.append(pltpu.async_copy(xs.at[srcbuf.at[b + j]], rows[j], gsem[j]))
            for j in range(NBUF):
                cps[j].wait()
                pltpu.async_copy(rows[j], acc.at[dstbuf.at[b + j]], ssem[j], add=True)
            return c
        lax.fori_loop(0, NB_W // NBUF, quad, 0)
        for j in range(NBUF):
            pltpu.make_async_copy(xs.at[pl.ds(0, EB)], rows[j], ssem[j]).wait()
        plsc.subcore_barrier()

        for j in range(RPT // 128):
            pltpu.sync_copy(acc.at[pl.ds(base + j * 128, 128)], rows[j % 2])
            pltpu.sync_copy(rows[j % 2], yout.at[cid, pl.ds(base + j * 128, 128)])
        plsc.subcore_barrier()


def _prep_body(d0, d1, x_ref, xs0, xs1, dinv_ref):
    deg = d0[...] + d1[...] + 1.0
    dinv = lax.rsqrt(deg)
    xs = x_ref[...] * dinv
    xs0[...] = xs[:, :128]
    xs1[...] = xs[:, 128:]
    dinv_ref[...] = dinv


def _dense_body(y0, y1, xs0, xs1, dinv, wbd, bias, attr, w1, w2, b1, b2, out_ref):
    i = pl.program_id(0)
    yl = y0[0] + y0[1] + xs0[...]
    yr = y1[0] + y1[1] + xs1[...]
    y = jnp.concatenate([yl, yr], axis=1) * dinv[...]
    g = jnp.dot(y, wbd[...], preferred_element_type=jnp.float32) + bias[...]
    z = jax.nn.sigmoid(g[:, :PERIODS * D])
    t = jnp.tanh(g[:, PERIODS * D:])
    p = jax.nn.softmax(attr[...], axis=1)
    h = jnp.zeros((BLK, D), jnp.float32)
    for tt in range(PERIODS):
        pt = p[:, tt:tt + 1]
        h = h + pt * (1.0 - z[:, tt * D:(tt + 1) * D]) * t[:, tt * D:(tt + 1) * D]
    v = jnp.dot(jnp.maximum(h, 0.0), w1[...], preferred_element_type=jnp.float32)
    ridx = lax.broadcasted_iota(jnp.int32, (BLK, 1), 0) + i * BLK
    valid = jnp.where(ridx < N, 1.0, 0.0)
    part = jnp.sum((v + b1[...]) * w2[...] * valid)

    @pl.when(i == 0)
    def _():
        out_ref[...] = b2[...]
    out_ref[...] += part


_sc_mesh = dict(core_axis_name="c", subcore_axis_name="s",
                num_cores=NC, num_subcores=NS)
_sc_params = pltpu.CompilerParams(needs_layout_passes=False)


def _deg_call(dstp):
    return pl.kernel(
        _deg_body,
        out_type=jax.ShapeDtypeStruct((NC, NROW128, 128), jnp.float32),
        mesh=plsc.VectorSubcoreMesh(**_sc_mesh),
        compiler_params=_sc_params,
        scratch_types=[
            pltpu.VMEM((NROW128, 128), jnp.float32),
            pltpu.VMEM((NB_W, EB), jnp.int32),
            pltpu.VMEM((5, 128), jnp.float32),
            pltpu.VMEM((NROW128,), jnp.int32),
            pltpu.VMEM_SHARED((NROW128, 128), jnp.float32),
        ],
    )(dstp)


def _agg_call(srcp, dstp, xs0, xs1):
    return pl.kernel(
        _agg_body,
        out_type=(
            jax.ShapeDtypeStruct((NC, NPAD, 128), jnp.float32),
            jax.ShapeDtypeStruct((NC, NPAD, 128), jnp.float32),
        ),
        mesh=plsc.VectorSubcoreMesh(**_sc_mesh),
        compiler_params=_sc_params,
        scratch_types=(
            [pltpu.VMEM((NB_W, EB), jnp.int32)] * 2 +
            [pltpu.VMEM((EB, 128), jnp.float32)] * NBUF +
            [pltpu.VMEM((1, 128), jnp.int32)] +
            [pltpu.SemaphoreType.DMA] * (2 * NBUF) +
            [pltpu.VMEM_SHARED((NPAD, 128), jnp.float32)]
        ),
    )(srcp, dstp, xs0, xs1)


def kernel(x, edge_index, att, Wcz, bcz, Wcr, bcr, Wch, bch,
           Wlz, blz, Wlr, blr, Wlh, blh, W1, b1, W2, b2):
    f32 = jnp.float32
    X = x.reshape(N, FP)
    Xp = jnp.pad(X, ((0, NPAD - N), (0, 0)))
    pad_e = EPAD - E
    srcp = jnp.concatenate(
        [edge_index[0], jnp.zeros((pad_e,), jnp.int32)]).reshape(NW * NB_W, EB)
    trash = N + (jnp.arange(pad_e, dtype=jnp.int32) % (NPAD - N))
    dstp = jnp.concatenate([edge_index[1], trash]).reshape(NW * NB_W, EB)

    Wz_eff = Wcz @ Wlz[:D]
    Wh_eff = Wch @ Wlh[:D]
    eye = jnp.eye(PERIODS, dtype=f32)
    WzBD = jnp.einsum("fd,pt->fptd", Wz_eff, eye).reshape(FP, PERIODS * D)
    WhBD = jnp.einsum("fd,pt->fptd", Wh_eff, eye).reshape(FP, PERIODS * D)
    WBD = jnp.concatenate([WzBD, WhBD], axis=1)
    bz = bcz @ Wlz[:D] + blz
    bh = bch @ Wlh[:D] + blh
    bias = jnp.concatenate([jnp.tile(bz, PERIODS),
                            jnp.tile(bh, PERIODS)]).reshape(1, 2 * PERIODS * D)
    W2p = jnp.pad(W2, ((0, NPAD - N), (0, 0)))
    attr = att.reshape(1, PERIODS)
    b1r = b1.reshape(1, 1)
    b2r = b2.reshape(1, 1)

    degp = _deg_call(dstp)
    d0 = degp[0].reshape(NPAD, 1)
    d1 = degp[1].reshape(NPAD, 1)

    xs0, xs1, dinv = pl.pallas_call(
        _prep_body,
        grid=(GRID,),
        in_specs=[
            pl.BlockSpec((BLK, 1), lambda i: (i, 0)),
            pl.BlockSpec((BLK, 1), lambda i: (i, 0)),
            pl.BlockSpec((BLK, FP), lambda i: (i, 0)),
        ],
        out_specs=[
            pl.BlockSpec((BLK, 128), lambda i: (i, 0)),
            pl.BlockSpec((BLK, 128), lambda i: (i, 0)),
            pl.BlockSpec((BLK, 1), lambda i: (i, 0)),
        ],
        out_shape=[
            jax.ShapeDtypeStruct((NPAD, 128), f32),
            jax.ShapeDtypeStruct((NPAD, 128), f32),
            jax.ShapeDtypeStruct((NPAD, 1), f32),
        ],
    )(d0, d1, Xp)

    y0, y1 = _agg_call(srcp, dstp, xs0, xs1)

    out = pl.pallas_call(
        _dense_body,
        grid=(GRID,),
        in_specs=[
            pl.BlockSpec((NC, BLK, 128), lambda i: (0, i, 0)),
            pl.BlockSpec((NC, BLK, 128), lambda i: (0, i, 0)),
            pl.BlockSpec((BLK, 128), lambda i: (i, 0)),
            pl.BlockSpec((BLK, 128), lambda i: (i, 0)),
            pl.BlockSpec((BLK, 1), lambda i: (i, 0)),
            pl.BlockSpec((FP, 4 * PERIODS * D // 2), lambda i: (0, 0)),
            pl.BlockSpec((1, 2 * PERIODS * D), lambda i: (0, 0)),
            pl.BlockSpec((1, PERIODS), lambda i: (0, 0)),
            pl.BlockSpec((D, 1), lambda i: (0, 0)),
            pl.BlockSpec((BLK, 1), lambda i: (i, 0)),
            pl.BlockSpec((1, 1), lambda i: (0, 0)),
            pl.BlockSpec((1, 1), lambda i: (0, 0)),
        ],
        out_specs=pl.BlockSpec((1, 1), lambda i: (0, 0)),
        out_shape=jax.ShapeDtypeStruct((1, 1), f32),
    )(y0, y1, xs0, xs1, dinv, WBD, bias, attr, W1, W2p, b1r, b2r)

    return out.reshape(-1)


# speedup vs baseline: 70.4375x; 1.0460x over previous
import jax
import jax.numpy as jnp
from jax import lax
from jax.experimental import pallas as pl
from jax.experimental.pallas import tpu as pltpu
from jax.experimental.pallas import tpu_sc as plsc

N = 10000
E = 160000
F_IN = 32
D = 256
PERIODS = 8
FP = F_IN * PERIODS

NC = 2
NS = 16
NW = NC * NS
NPAD = 10240
NROW128 = NPAD // 128
EB = 128
NB_W = 40
EPW = NB_W * EB
EPAD = NW * EPW
TRASH = N + 100
RPT = NPAD // NS
BLK = 256
GRID = NPAD // BLK


def _zero2d(ref, nrows):
    def body(r, c):
        for j in range(8):
            ref[r, pl.ds(j * 16, 16)] = jnp.zeros((16,), jnp.float32)
        return c
    lax.fori_loop(0, nrows, body, 0)


def _deg_body(edges_hbm, deg_out, hist, dstbuf, zrows, rowidx, acc):
    cid = lax.axis_index("c")
    sid = lax.axis_index("s")
    w = cid * NS + sid
    _zero2d(hist, NROW128)
    pltpu.sync_copy(edges_hbm.at[pl.ds(w * NB_W, NB_W)], dstbuf)

    ones = jnp.ones((16,), jnp.float32)

    def hb(j, c):
        r = j >> 3
        k = j & 7
        idx = dstbuf[r, pl.ds(k * 16, 16)] & 0xFFFF
        plsc.addupdate_scatter(hist, [idx >> 7, idx & 127], ones)
        return c
    lax.fori_loop(0, NB_W * 8, hb, 0)

    _zero2d(zrows, 5)
    pltpu.sync_copy(zrows, acc.at[pl.ds(sid * 5, 5)])
    plsc.subcore_barrier()

    def ri(i, c):
        rowidx[pl.ds(i * 16, 16)] = lax.iota(jnp.int32, 16) + i * 16
        return c
    lax.fori_loop(0, NROW128 // 16, ri, 0)
    pltpu.sync_copy(hist, acc.at[rowidx], add=True)
    plsc.subcore_barrier()

    @pl.when(sid == 0)
    def _():
        pltpu.sync_copy(acc, hist)
        pltpu.sync_copy(hist, deg_out.at[cid])


NBUF = 2
NB_A = 16
NB_B = 64
NBMAX = max(NB_A, NB_B)


def _agg_body(edges_hbm, xs0, xs1, y0_out, y1_out,
              pbuf, si0, si1, di0, di1, r0, r1,
              g0, g1, s0, s1, acc):
    rows = (r0, r1)
    sidx = (si0, si1)
    didx = (di0, di1)
    gsem = (g0, g1)
    ssem = (s0, s1)
    cid = lax.axis_index("c")
    sid = lax.axis_index("s")
    nb = jnp.where(cid == 0, NB_A, NB_B)
    row0 = jnp.where(cid == 0, sid * NB_A, NS * NB_A + sid * NB_B)
    pltpu.sync_copy(edges_hbm.at[pl.ds(row0, NBMAX)], pbuf)
    base = sid * RPT

    for chunk in range(2):
        xs = (xs0, xs1)[chunk]
        yout = (y0_out, y1_out)[chunk]
        _zero2d(r0, 128)
        for j in range(RPT // 128):
            pltpu.sync_copy(r0, acc.at[pl.ds(base + j * 128, 128)])
        plsc.subcore_barrier()

        def quad(q, c):
            b = q * NBUF
            cps = []
            for j in range(NBUF):
                @pl.when(q > 0)
                def _():
                    pltpu.make_async_copy(xs.at[pl.ds(0, EB)], rows[j],
                                          ssem[j]).wait()
                for k in range(EB // 16):
                    pk = pbuf[b + j, pl.ds(k * 16, 16)]
                    sidx[j][0, pl.ds(k * 16, 16)] = pk >> 16
                    didx[j][0, pl.ds(k * 16, 16)] = pk & 0xFFFF
                cps.append(pltpu.async_copy(xs.at[sidx[j].at[0]], rows[j],
                                            gsem[j]))
            for j in range(NBUF):
                cps[j].wait()
                pltpu.async_copy(rows[j], acc.at[didx[j].at[0]], ssem[j],
                                 add=True)
            return c
        lax.fori_loop(0, nb // NBUF, quad, 0)
        for j in range(NBUF):
            pltpu.make_async_copy(xs.at[pl.ds(0, EB)], rows[j], ssem[j]).wait()
        plsc.subcore_barrier()

        for j in range(RPT // 128):
            pltpu.sync_copy(acc.at[pl.ds(base + j * 128, 128)], rows[j % 2])
            pltpu.sync_copy(rows[j % 2], yout.at[cid, pl.ds(base + j * 128, 128)])
        plsc.subcore_barrier()


def _prep_body(d0, d1, x_ref, xs0, xs1, dinv_ref):
    deg = d0[...] + d1[...] + 1.0
    dinv = lax.rsqrt(deg)
    xs = x_ref[...] * dinv
    xs0[...] = xs[:, :128]
    xs1[...] = xs[:, 128:]
    dinv_ref[...] = dinv


def _dense_body(y0, y1, xs0, xs1, dinv, wbd, bias, attr, w1, w2, b1, b2, out_ref):
    i = pl.program_id(0)
    yl = y0[0] + y0[1] + xs0[...]
    yr = y1[0] + y1[1] + xs1[...]
    y = jnp.concatenate([yl, yr], axis=1) * dinv[...]
    g = jnp.dot(y, wbd[...], preferred_element_type=jnp.float32) + bias[...]
    z = jax.nn.sigmoid(g[:, :PERIODS * D])
    t = jnp.tanh(g[:, PERIODS * D:])
    p = jax.nn.softmax(attr[...], axis=1)
    h = jnp.zeros((BLK, D), jnp.float32)
    for tt in range(PERIODS):
        pt = p[:, tt:tt + 1]
        h = h + pt * (1.0 - z[:, tt * D:(tt + 1) * D]) * t[:, tt * D:(tt + 1) * D]
    v = jnp.dot(jnp.maximum(h, 0.0), w1[...], preferred_element_type=jnp.float32)
    ridx = lax.broadcasted_iota(jnp.int32, (BLK, 1), 0) + i * BLK
    valid = jnp.where(ridx < N, 1.0, 0.0)
    part = jnp.sum((v + b1[...]) * w2[...] * valid)

    @pl.when(i == 0)
    def _():
        out_ref[...] = b2[...]
    out_ref[...] += part


_sc_mesh = dict(core_axis_name="c", subcore_axis_name="s",
                num_cores=NC, num_subcores=NS)
_sc_params = pltpu.CompilerParams(needs_layout_passes=False)


def _deg_call(dstp):
    return pl.kernel(
        _deg_body,
        out_type=jax.ShapeDtypeStruct((NC, NROW128, 128), jnp.float32),
        mesh=plsc.VectorSubcoreMesh(**_sc_mesh),
        compiler_params=_sc_params,
        scratch_types=[
            pltpu.VMEM((NROW128, 128), jnp.float32),
            pltpu.VMEM((NB_W, EB), jnp.int32),
            pltpu.VMEM((5, 128), jnp.float32),
            pltpu.VMEM((NROW128,), jnp.int32),
            pltpu.VMEM_SHARED((NROW128, 128), jnp.float32),
        ],
    )(dstp)


def _agg_call(edges, xs0, xs1):
    return pl.kernel(
        _agg_body,
        out_type=(
            jax.ShapeDtypeStruct((NC, NPAD, 128), jnp.float32),
            jax.ShapeDtypeStruct((NC, NPAD, 128), jnp.float32),
        ),
        mesh=plsc.VectorSubcoreMesh(**_sc_mesh),
        compiler_params=_sc_params,
        scratch_types=(
            [pltpu.VMEM((NBMAX, EB), jnp.int32)] +
            [pltpu.VMEM((1, EB), jnp.int32)] * 4 +
            [pltpu.VMEM((EB, 128), jnp.float32)] * NBUF +
            [pltpu.SemaphoreType.DMA] * (2 * NBUF) +
            [pltpu.VMEM_SHARED((NPAD, 128), jnp.float32)]
        ),
    )(edges, xs0, xs1)


def kernel(x, edge_index, att, Wcz, bcz, Wcr, bcr, Wch, bch,
           Wlz, blz, Wlr, blr, Wlh, blh, W1, b1, W2, b2):
    f32 = jnp.float32
    X = x.reshape(N, FP)
    Xp = jnp.pad(X, ((0, NPAD - N), (0, 0)))
    pad_e = EPAD - E
    srcp = jnp.concatenate([edge_index[0], jnp.zeros((pad_e,), jnp.int32)])
    trash = N + (jnp.arange(pad_e, dtype=jnp.int32) % (NPAD - N))
    dstp = jnp.concatenate([edge_index[1], trash])
    edges = ((srcp << 16) | dstp).reshape(NW * NB_W, EB)

    Wz_eff = Wcz @ Wlz[:D]
    Wh_eff = Wch @ Wlh[:D]
    eye = jnp.eye(PERIODS, dtype=f32)
    WzBD = jnp.einsum("fd,pt->fptd", Wz_eff, eye).reshape(FP, PERIODS * D)
    WhBD = jnp.einsum("fd,pt->fptd", Wh_eff, eye).reshape(FP, PERIODS * D)
    WBD = jnp.concatenate([WzBD, WhBD], axis=1)
    bz = bcz @ Wlz[:D] + blz
    bh = bch @ Wlh[:D] + blh
    bias = jnp.concatenate([jnp.tile(bz, PERIODS),
                            jnp.tile(bh, PERIODS)]).reshape(1, 2 * PERIODS * D)
    W2p = jnp.pad(W2, ((0, NPAD - N), (0, 0)))
    attr = att.reshape(1, PERIODS)
    b1r = b1.reshape(1, 1)
    b2r = b2.reshape(1, 1)

    degp = _deg_call(edges)
    d0 = degp[0].reshape(NPAD, 1)
    d1 = degp[1].reshape(NPAD, 1)

    xs0, xs1, dinv = pl.pallas_call(
        _prep_body,
        grid=(GRID,),
        in_specs=[
            pl.BlockSpec((BLK, 1), lambda i: (i, 0)),
            pl.BlockSpec((BLK, 1), lambda i: (i, 0)),
            pl.BlockSpec((BLK, FP), lambda i: (i, 0)),
        ],
        out_specs=[
            pl.BlockSpec((BLK, 128), lambda i: (i, 0)),
            pl.BlockSpec((BLK, 128), lambda i: (i, 0)),
            pl.BlockSpec((BLK, 1), lambda i: (i, 0)),
        ],
        out_shape=[
            jax.ShapeDtypeStruct((NPAD, 128), f32),
            jax.ShapeDtypeStruct((NPAD, 128), f32),
            jax.ShapeDtypeStruct((NPAD, 1), f32),
        ],
    )(d0, d1, Xp)

    y0, y1 = _agg_call(edges, xs0, xs1)

    out = pl.pallas_call(
        _dense_body,
        grid=(GRID,),
        in_specs=[
            pl.BlockSpec((NC, BLK, 128), lambda i: (0, i, 0)),
            pl.BlockSpec((NC, BLK, 128), lambda i: (0, i, 0)),
            pl.BlockSpec((BLK, 128), lambda i: (i, 0)),
            pl.BlockSpec((BLK, 128), lambda i: (i, 0)),
            pl.BlockSpec((BLK, 1), lambda i: (i, 0)),
            pl.BlockSpec((FP, 4 * PERIODS * D // 2), lambda i: (0, 0)),
            pl.BlockSpec((1, 2 * PERIODS * D), lambda i: (0, 0)),
            pl.BlockSpec((1, PERIODS), lambda i: (0, 0)),
            pl.BlockSpec((D, 1), lambda i: (0, 0)),
            pl.BlockSpec((BLK, 1), lambda i: (i, 0)),
            pl.BlockSpec((1, 1), lambda i: (0, 0)),
            pl.BlockSpec((1, 1), lambda i: (0, 0)),
        ],
        out_specs=pl.BlockSpec((1, 1), lambda i: (0, 0)),
        out_shape=jax.ShapeDtypeStruct((1, 1), f32),
    )(y0, y1, xs0, xs1, dinv, WBD, bias, attr, W1, W2p, b1r, b2r)

    return out.reshape(-1)


# speedup vs baseline: 70.6030x; 1.0023x over previous
import jax
import jax.numpy as jnp
from jax import lax
from jax.experimental import pallas as pl
from jax.experimental.pallas import tpu as pltpu
from jax.experimental.pallas import tpu_sc as plsc

N = 10000
E = 160000
F_IN = 32
D = 256
PERIODS = 8
FP = F_IN * PERIODS

NC = 2
NS = 16
NW = NC * NS
NPAD = 10240
NROW128 = NPAD // 128
EB = 128
NB_W = 40
EPW = NB_W * EB
EPAD = NW * EPW
TRASH = N + 100
RPT = NPAD // NS
BLK = 256
GRID = NPAD // BLK


def _zero2d(ref, nrows):
    def body(r, c):
        for j in range(8):
            ref[r, pl.ds(j * 16, 16)] = jnp.zeros((16,), jnp.float32)
        return c
    lax.fori_loop(0, nrows, body, 0)


def _deg_body(edges_hbm, deg_out, hist, dstbuf, zrows, rowidx, acc):
    cid = lax.axis_index("c")
    sid = lax.axis_index("s")
    w = cid * NS + sid
    _zero2d(hist, NROW128)
    pltpu.sync_copy(edges_hbm.at[pl.ds(w * NB_W, NB_W)], dstbuf)

    ones = jnp.ones((16,), jnp.float32)

    def hb(j, c):
        r = j >> 3
        k = j & 7
        idx = dstbuf[r, pl.ds(k * 16, 16)] & 0xFFFF
        plsc.addupdate_scatter(hist, [idx >> 7, idx & 127], ones)
        return c
    lax.fori_loop(0, NB_W * 8, hb, 0)

    _zero2d(zrows, 5)
    pltpu.sync_copy(zrows, acc.at[pl.ds(sid * 5, 5)])
    plsc.subcore_barrier()

    def ri(i, c):
        rowidx[pl.ds(i * 16, 16)] = lax.iota(jnp.int32, 16) + i * 16
        return c
    lax.fori_loop(0, NROW128 // 16, ri, 0)
    pltpu.sync_copy(hist, acc.at[rowidx], add=True)
    plsc.subcore_barrier()

    @pl.when(sid == 0)
    def _():
        pltpu.sync_copy(acc, hist)
        pltpu.sync_copy(hist, deg_out.at[cid])


NBUF = 2
NB_A = 16
NB_B = 64
NBMAX = max(NB_A, NB_B)


def _agg_body(edges_hbm, xs0, xs1, y0_out, y1_out,
              pbuf, si0, si1, di0, di1, r0, r1,
              g0, g1, s0, s1, acc):
    rows = (r0, r1)
    sidx = (si0, si1)
    didx = (di0, di1)
    gsem = (g0, g1)
    ssem = (s0, s1)
    cid = lax.axis_index("c")
    sid = lax.axis_index("s")
    nb = jnp.where(cid == 0, NB_A, NB_B)
    row0 = jnp.where(cid == 0, sid * NB_A, NS * NB_A + sid * NB_B)
    pltpu.sync_copy(edges_hbm.at[pl.ds(row0, NBMAX)], pbuf)
    base = sid * RPT

    for chunk in range(2):
        xs = (xs0, xs1)[chunk]
        yout = (y0_out, y1_out)[chunk]
        with jax.named_scope("zerofill"):
            _zero2d(r0, 128)
            for j in range(RPT // 128):
                pltpu.sync_copy(r0, acc.at[pl.ds(base + j * 128, 128)])
            plsc.subcore_barrier()

        def quad(q, c):
            b = q * NBUF
            cps = []
            for j in range(NBUF):
                @pl.when(q > 0)
                def _():
                    pltpu.make_async_copy(xs.at[pl.ds(0, EB)], rows[j],
                                          ssem[j]).wait()
                for k in range(EB // 16):
                    pk = pbuf[b + j, pl.ds(k * 16, 16)]
                    sidx[j][0, pl.ds(k * 16, 16)] = pk >> 16
                    didx[j][0, pl.ds(k * 16, 16)] = pk & 0xFFFF
                cps.append(pltpu.async_copy(xs.at[sidx[j].at[0]], rows[j],
                                            gsem[j]))
            for j in range(NBUF):
                cps[j].wait()
                pltpu.async_copy(rows[j], acc.at[didx[j].at[0]], ssem[j],
                                 add=True)
            return c
        with jax.named_scope("edges"):
            lax.fori_loop(0, nb // NBUF, quad, 0)
            for j in range(NBUF):
                pltpu.make_async_copy(xs.at[pl.ds(0, EB)], rows[j],
                                      ssem[j]).wait()
            plsc.subcore_barrier()

        with jax.named_scope("writeout"):
            for j in range(RPT // 128):
                pltpu.sync_copy(acc.at[pl.ds(base + j * 128, 128)], rows[j % 2])
                pltpu.sync_copy(rows[j % 2],
                                yout.at[cid, pl.ds(base + j * 128, 128)])
            plsc.subcore_barrier()


def _prep_body(d0, d1, x_ref, xs0, xs1, dinv_ref):
    deg = d0[...] + d1[...] + 1.0
    dinv = lax.rsqrt(deg)
    xs = x_ref[...] * dinv
    xs0[...] = xs[:, :128]
    xs1[...] = xs[:, 128:]
    dinv_ref[...] = dinv


def _dense_body(y0, y1, xs0, xs1, dinv, wbd, bias, attr, w1, w2, b1, b2, out_ref):
    i = pl.program_id(0)
    yl = y0[0] + y0[1] + xs0[...]
    yr = y1[0] + y1[1] + xs1[...]
    y = jnp.concatenate([yl, yr], axis=1) * dinv[...]
    g = jnp.dot(y, wbd[...], preferred_element_type=jnp.float32) + bias[...]
    z = jax.nn.sigmoid(g[:, :PERIODS * D])
    t = jnp.tanh(g[:, PERIODS * D:])
    p = jax.nn.softmax(attr[...], axis=1)
    h = jnp.zeros((BLK, D), jnp.float32)
    for tt in range(PERIODS):
        pt = p[:, tt:tt + 1]
        h = h + pt * (1.0 - z[:, tt * D:(tt + 1) * D]) * t[:, tt * D:(tt + 1) * D]
    v = jnp.dot(jnp.maximum(h, 0.0), w1[...], preferred_element_type=jnp.float32)
    ridx = lax.broadcasted_iota(jnp.int32, (BLK, 1), 0) + i * BLK
    valid = jnp.where(ridx < N, 1.0, 0.0)
    part = jnp.sum((v + b1[...]) * w2[...] * valid)

    @pl.when(i == 0)
    def _():
        out_ref[...] = b2[...]
    out_ref[...] += part


_sc_mesh = dict(core_axis_name="c", subcore_axis_name="s",
                num_cores=NC, num_subcores=NS)
_sc_params = pltpu.CompilerParams(needs_layout_passes=False)


def _deg_call(dstp):
    return pl.kernel(
        _deg_body,
        out_type=jax.ShapeDtypeStruct((NC, NROW128, 128), jnp.float32),
        mesh=plsc.VectorSubcoreMesh(**_sc_mesh),
        compiler_params=_sc_params,
        scratch_types=[
            pltpu.VMEM((NROW128, 128), jnp.float32),
            pltpu.VMEM((NB_W, EB), jnp.int32),
            pltpu.VMEM((5, 128), jnp.float32),
            pltpu.VMEM((NROW128,), jnp.int32),
            pltpu.VMEM_SHARED((NROW128, 128), jnp.float32),
        ],
    )(dstp)


def _agg_call(edges, xs0, xs1):
    return pl.kernel(
        _agg_body,
        out_type=(
            jax.ShapeDtypeStruct((NC, NPAD, 128), jnp.float32),
            jax.ShapeDtypeStruct((NC, NPAD, 128), jnp.float32),
        ),
        mesh=plsc.VectorSubcoreMesh(**_sc_mesh),
        compiler_params=_sc_params,
        scratch_types=(
            [pltpu.VMEM((NBMAX, EB), jnp.int32)] +
            [pltpu.VMEM((1, EB), jnp.int32)] * 4 +
            [pltpu.VMEM((EB, 128), jnp.float32)] * NBUF +
            [pltpu.SemaphoreType.DMA] * (2 * NBUF) +
            [pltpu.VMEM_SHARED((NPAD, 128), jnp.float32)]
        ),
    )(edges, xs0, xs1)


def kernel(x, edge_index, att, Wcz, bcz, Wcr, bcr, Wch, bch,
           Wlz, blz, Wlr, blr, Wlh, blh, W1, b1, W2, b2):
    f32 = jnp.float32
    X = x.reshape(N, FP)
    Xp = jnp.pad(X, ((0, NPAD - N), (0, 0)))
    pad_e = EPAD - E
    srcp = jnp.concatenate([edge_index[0], jnp.zeros((pad_e,), jnp.int32)])
    trash = N + (jnp.arange(pad_e, dtype=jnp.int32) % (NPAD - N))
    dstp = jnp.concatenate([edge_index[1], trash])
    edges = ((srcp << 16) | dstp).reshape(NW * NB_W, EB)

    Wz_eff = Wcz @ Wlz[:D]
    Wh_eff = Wch @ Wlh[:D]
    eye = jnp.eye(PERIODS, dtype=f32)
    WzBD = jnp.einsum("fd,pt->fptd", Wz_eff, eye).reshape(FP, PERIODS * D)
    WhBD = jnp.einsum("fd,pt->fptd", Wh_eff, eye).reshape(FP, PERIODS * D)
    WBD = jnp.concatenate([WzBD, WhBD], axis=1)
    bz = bcz @ Wlz[:D] + blz
    bh = bch @ Wlh[:D] + blh
    bias = jnp.concatenate([jnp.tile(bz, PERIODS),
                            jnp.tile(bh, PERIODS)]).reshape(1, 2 * PERIODS * D)
    W2p = jnp.pad(W2, ((0, NPAD - N), (0, 0)))
    attr = att.reshape(1, PERIODS)
    b1r = b1.reshape(1, 1)
    b2r = b2.reshape(1, 1)

    degp = _deg_call(edges)
    d0 = degp[0].reshape(NPAD, 1)
    d1 = degp[1].reshape(NPAD, 1)

    xs0, xs1, dinv = pl.pallas_call(
        _prep_body,
        grid=(GRID,),
        in_specs=[
            pl.BlockSpec((BLK, 1), lambda i: (i, 0)),
            pl.BlockSpec((BLK, 1), lambda i: (i, 0)),
            pl.BlockSpec((BLK, FP), lambda i: (i, 0)),
        ],
        out_specs=[
            pl.BlockSpec((BLK, 128), lambda i: (i, 0)),
            pl.BlockSpec((BLK, 128), lambda i: (i, 0)),
            pl.BlockSpec((BLK, 1), lambda i: (i, 0)),
        ],
        out_shape=[
            jax.ShapeDtypeStruct((NPAD, 128), f32),
            jax.ShapeDtypeStruct((NPAD, 128), f32),
            jax.ShapeDtypeStruct((NPAD, 1), f32),
        ],
    )(d0, d1, Xp)

    y0, y1 = _agg_call(edges, xs0, xs1)

    out = pl.pallas_call(
        _dense_body,
        grid=(GRID,),
        in_specs=[
            pl.BlockSpec((NC, BLK, 128), lambda i: (0, i, 0)),
            pl.BlockSpec((NC, BLK, 128), lambda i: (0, i, 0)),
            pl.BlockSpec((BLK, 128), lambda i: (i, 0)),
            pl.BlockSpec((BLK, 128), lambda i: (i, 0)),
            pl.BlockSpec((BLK, 1), lambda i: (i, 0)),
            pl.BlockSpec((FP, 4 * PERIODS * D // 2), lambda i: (0, 0)),
            pl.BlockSpec((1, 2 * PERIODS * D), lambda i: (0, 0)),
            pl.BlockSpec((1, PERIODS), lambda i: (0, 0)),
            pl.BlockSpec((D, 1), lambda i: (0, 0)),
            pl.BlockSpec((BLK, 1), lambda i: (i, 0)),
            pl.BlockSpec((1, 1), lambda i: (0, 0)),
            pl.BlockSpec((1, 1), lambda i: (0, 0)),
        ],
        out_specs=pl.BlockSpec((1, 1), lambda i: (0, 0)),
        out_shape=jax.ShapeDtypeStruct((1, 1), f32),
    )(y0, y1, xs0, xs1, dinv, WBD, bias, attr, W1, W2p, b1r, b2r)

    return out.reshape(-1)


# speedup vs baseline: 84.4027x; 1.1955x over previous
import jax
import jax.numpy as jnp
from jax import lax
from jax.experimental import pallas as pl
from jax.experimental.pallas import tpu as pltpu
from jax.experimental.pallas import tpu_sc as plsc

N = 10000
E = 160000
F_IN = 32
D = 256
PERIODS = 8
FP = F_IN * PERIODS

NC = 2
NS = 16
NW = NC * NS
NPAD = 10240
NROW128 = NPAD // 128
EB = 128
NB_W = 40
EPW = NB_W * EB
EPAD = NW * EPW
TRASH = N + 100
RPT = NPAD // NS
BLK = 256
GRID = NPAD // BLK


def _zero2d(ref, nrows):
    def body(r, c):
        for j in range(8):
            ref[r, pl.ds(j * 16, 16)] = jnp.zeros((16,), jnp.float32)
        return c
    lax.fori_loop(0, nrows, body, 0)


def _deg_body(edges_hbm, deg_out, hist, dstbuf, zrows, rowidx, acc):
    cid = lax.axis_index("c")
    sid = lax.axis_index("s")
    w = cid * NS + sid
    _zero2d(hist, NROW128)
    pltpu.sync_copy(edges_hbm.at[pl.ds(w * NB_W, NB_W)], dstbuf)

    ones = jnp.ones((16,), jnp.float32)

    def hb(j, c):
        r = j >> 3
        k = j & 7
        idx = dstbuf[r, pl.ds(k * 16, 16)] & 0xFFFF
        plsc.addupdate_scatter(hist, [idx >> 7, idx & 127], ones)
        return c
    lax.fori_loop(0, NB_W * 8, hb, 0)

    _zero2d(zrows, 5)
    pltpu.sync_copy(zrows, acc.at[pl.ds(sid * 5, 5)])
    plsc.subcore_barrier()

    def ri(i, c):
        rowidx[pl.ds(i * 16, 16)] = lax.iota(jnp.int32, 16) + i * 16
        return c
    lax.fori_loop(0, NROW128 // 16, ri, 0)
    pltpu.sync_copy(hist, acc.at[rowidx], add=True)
    plsc.subcore_barrier()

    @pl.when(sid == 0)
    def _():
        pltpu.sync_copy(acc, hist)
        pltpu.sync_copy(hist, deg_out.at[cid])


NBUF = 2
NB_A = 56
NB_B = 24
NBMAX = max(NB_A, NB_B)


def _agg_body(edges_hbm, xs0, xs1, y0_out, y1_out,
              pbuf, si0, si1, di0, di1, r0, r1,
              g0, g1, s0, s1, acc):
    rows = (r0, r1)
    sidx = (si0, si1)
    didx = (di0, di1)
    gsem = (g0, g1)
    ssem = (s0, s1)
    cid = lax.axis_index("c")
    sid = lax.axis_index("s")
    nb = jnp.where(cid == 0, NB_A, NB_B)
    row0 = jnp.where(cid == 0, sid * NB_A, NS * NB_A + sid * NB_B)
    pltpu.sync_copy(edges_hbm.at[pl.ds(row0, NBMAX)], pbuf)
    base = sid * RPT

    for chunk in range(2):
        xs = (xs0, xs1)[chunk]
        yout = (y0_out, y1_out)[chunk]
        with jax.named_scope("zerofill"):
            _zero2d(r0, 128)
            for j in range(RPT // 128):
                pltpu.sync_copy(r0, acc.at[pl.ds(base + j * 128, 128)])
            plsc.subcore_barrier()

        def quad(q, c):
            b = q * NBUF
            cps = []
            for j in range(NBUF):
                @pl.when(q > 0)
                def _():
                    pltpu.make_async_copy(xs.at[pl.ds(0, EB)], rows[j],
                                          ssem[j]).wait()
                for k in range(EB // 16):
                    pk = pbuf[b + j, pl.ds(k * 16, 16)]
                    sidx[j][0, pl.ds(k * 16, 16)] = pk >> 16
                    didx[j][0, pl.ds(k * 16, 16)] = pk & 0xFFFF
                cps.append(pltpu.async_copy(xs.at[sidx[j].at[0]], rows[j],
                                            gsem[j]))
            for j in range(NBUF):
                cps[j].wait()
                pltpu.async_copy(rows[j], acc.at[didx[j].at[0]], ssem[j],
                                 add=True)
            return c
        with jax.named_scope("edges"):
            lax.fori_loop(0, nb // NBUF, quad, 0)
            for j in range(NBUF):
                pltpu.make_async_copy(xs.at[pl.ds(0, EB)], rows[j],
                                      ssem[j]).wait()
            plsc.subcore_barrier()

        with jax.named_scope("writeout"):
            for j in range(RPT // 128):
                pltpu.sync_copy(acc.at[pl.ds(base + j * 128, 128)], rows[j % 2])
                pltpu.sync_copy(rows[j % 2],
                                yout.at[cid, pl.ds(base + j * 128, 128)])
            plsc.subcore_barrier()


def _prep_body(d0, d1, x_ref, xs0, xs1, dinv_ref):
    deg = d0[...] + d1[...] + 1.0
    dinv = lax.rsqrt(deg)
    xs = x_ref[...] * dinv
    xs0[...] = xs[:, :128]
    xs1[...] = xs[:, 128:]
    dinv_ref[...] = dinv


def _dense_body(y0, y1, xs0, xs1, dinv, wbd, bias, attr, w1, w2, b1, b2, out_ref):
    i = pl.program_id(0)
    yl = y0[0] + y0[1] + xs0[...]
    yr = y1[0] + y1[1] + xs1[...]
    y = jnp.concatenate([yl, yr], axis=1) * dinv[...]
    g = jnp.dot(y, wbd[...], preferred_element_type=jnp.float32) + bias[...]
    z = jax.nn.sigmoid(g[:, :PERIODS * D])
    t = jnp.tanh(g[:, PERIODS * D:])
    p = jax.nn.softmax(attr[...], axis=1)
    h = jnp.zeros((BLK, D), jnp.float32)
    for tt in range(PERIODS):
        pt = p[:, tt:tt + 1]
        h = h + pt * (1.0 - z[:, tt * D:(tt + 1) * D]) * t[:, tt * D:(tt + 1) * D]
    v = jnp.dot(jnp.maximum(h, 0.0), w1[...], preferred_element_type=jnp.float32)
    ridx = lax.broadcasted_iota(jnp.int32, (BLK, 1), 0) + i * BLK
    valid = jnp.where(ridx < N, 1.0, 0.0)
    part = jnp.sum((v + b1[...]) * w2[...] * valid)

    @pl.when(i == 0)
    def _():
        out_ref[...] = b2[...]
    out_ref[...] += part


_sc_mesh = dict(core_axis_name="c", subcore_axis_name="s",
                num_cores=NC, num_subcores=NS)
_sc_params = pltpu.CompilerParams(needs_layout_passes=False)


def _deg_call(dstp):
    return pl.kernel(
        _deg_body,
        out_type=jax.ShapeDtypeStruct((NC, NROW128, 128), jnp.float32),
        mesh=plsc.VectorSubcoreMesh(**_sc_mesh),
        compiler_params=_sc_params,
        scratch_types=[
            pltpu.VMEM((NROW128, 128), jnp.float32),
            pltpu.VMEM((NB_W, EB), jnp.int32),
            pltpu.VMEM((5, 128), jnp.float32),
            pltpu.VMEM((NROW128,), jnp.int32),
            pltpu.VMEM_SHARED((NROW128, 128), jnp.float32),
        ],
    )(dstp)


def _agg_call(edges, xs0, xs1):
    return pl.kernel(
        _agg_body,
        out_type=(
            jax.ShapeDtypeStruct((NC, NPAD, 128), jnp.float32),
            jax.ShapeDtypeStruct((NC, NPAD, 128), jnp.float32),
        ),
        mesh=plsc.VectorSubcoreMesh(**_sc_mesh),
        compiler_params=_sc_params,
        scratch_types=(
            [pltpu.VMEM((NBMAX, EB), jnp.int32)] +
            [pltpu.VMEM((1, EB), jnp.int32)] * 4 +
            [pltpu.VMEM((EB, 128), jnp.float32)] * NBUF +
            [pltpu.SemaphoreType.DMA] * (2 * NBUF) +
            [pltpu.VMEM_SHARED((NPAD, 128), jnp.float32)]
        ),
    )(edges, xs0, xs1)


def kernel(x, edge_index, att, Wcz, bcz, Wcr, bcr, Wch, bch,
           Wlz, blz, Wlr, blr, Wlh, blh, W1, b1, W2, b2):
    f32 = jnp.float32
    X = x.reshape(N, FP)
    Xp = jnp.pad(X, ((0, NPAD - N), (0, 0)))
    pad_e = EPAD - E
    srcp = jnp.concatenate([edge_index[0], jnp.zeros((pad_e,), jnp.int32)])
    trash = N + (jnp.arange(pad_e, dtype=jnp.int32) % (NPAD - N))
    dstp = jnp.concatenate([edge_index[1], trash])
    edges = ((srcp << 16) | dstp).reshape(NW * NB_W, EB)

    Wz_eff = Wcz @ Wlz[:D]
    Wh_eff = Wch @ Wlh[:D]
    eye = jnp.eye(PERIODS, dtype=f32)
    WzBD = jnp.einsum("fd,pt->fptd", Wz_eff, eye).reshape(FP, PERIODS * D)
    WhBD = jnp.einsum("fd,pt->fptd", Wh_eff, eye).reshape(FP, PERIODS * D)
    WBD = jnp.concatenate([WzBD, WhBD], axis=1)
    bz = bcz @ Wlz[:D] + blz
    bh = bch @ Wlh[:D] + blh
    bias = jnp.concatenate([jnp.tile(bz, PERIODS),
                            jnp.tile(bh, PERIODS)]).reshape(1, 2 * PERIODS * D)
    W2p = jnp.pad(W2, ((0, NPAD - N), (0, 0)))
    attr = att.reshape(1, PERIODS)
    b1r = b1.reshape(1, 1)
    b2r = b2.reshape(1, 1)

    degp = _deg_call(edges)
    d0 = degp[0].reshape(NPAD, 1)
    d1 = degp[1].reshape(NPAD, 1)

    xs0, xs1, dinv = pl.pallas_call(
        _prep_body,
        grid=(GRID,),
        in_specs=[
            pl.BlockSpec((BLK, 1), lambda i: (i, 0)),
            pl.BlockSpec((BLK, 1), lambda i: (i, 0)),
            pl.BlockSpec((BLK, FP), lambda i: (i, 0)),
        ],
        out_specs=[
            pl.BlockSpec((BLK, 128), lambda i: (i, 0)),
            pl.BlockSpec((BLK, 128), lambda i: (i, 0)),
            pl.BlockSpec((BLK, 1), lambda i: (i, 0)),
        ],
        out_shape=[
            jax.ShapeDtypeStruct((NPAD, 128), f32),
            jax.ShapeDtypeStruct((NPAD, 128), f32),
            jax.ShapeDtypeStruct((NPAD, 1), f32),
        ],
    )(d0, d1, Xp)

    y0, y1 = _agg_call(edges, xs0, xs1)

    out = pl.pallas_call(
        _dense_body,
        grid=(GRID,),
        in_specs=[
            pl.BlockSpec((NC, BLK, 128), lambda i: (0, i, 0)),
            pl.BlockSpec((NC, BLK, 128), lambda i: (0, i, 0)),
            pl.BlockSpec((BLK, 128), lambda i: (i, 0)),
            pl.BlockSpec((BLK, 128), lambda i: (i, 0)),
            pl.BlockSpec((BLK, 1), lambda i: (i, 0)),
            pl.BlockSpec((FP, 4 * PERIODS * D // 2), lambda i: (0, 0)),
            pl.BlockSpec((1, 2 * PERIODS * D), lambda i: (0, 0)),
            pl.BlockSpec((1, PERIODS), lambda i: (0, 0)),
            pl.BlockSpec((D, 1), lambda i: (0, 0)),
            pl.BlockSpec((BLK, 1), lambda i: (i, 0)),
            pl.BlockSpec((1, 1), lambda i: (0, 0)),
            pl.BlockSpec((1, 1), lambda i: (0, 0)),
        ],
        out_specs=pl.BlockSpec((1, 1), lambda i: (0, 0)),
        out_shape=jax.ShapeDtypeStruct((1, 1), f32),
    )(y0, y1, xs0, xs1, dinv, WBD, bias, attr, W1, W2p, b1r, b2r)

    return out.reshape(-1)
